# Initial kernel scaffold; baseline (speedup 1.0000x reference)
import jax
import jax.numpy as jnp
from jax.experimental import pallas as pl


def kernel(x, edge_index, W_in, b_in, W_conv, b_conv, gamma, beta, W_out, b_out):
    raise NotImplementedError("write your pallas kernel here")


# speedup vs baseline: 13.4075x; 13.4075x over previous
import functools

import jax
import jax.numpy as jnp
from jax import lax
from jax.experimental import pallas as pl
from jax.experimental.pallas import tpu as pltpu
from jax.experimental.pallas import tpu_sc as plsc

EPS = 1e-5
F32 = jnp.float32
CH = 128
NCORES = 2
NSUB = 16
NTILES = NCORES * NSUB


def _build_sc_scatter(n_pad, d, ncht, cpt):
    rpt = n_pad // NSUB
    grp = cpt // 2

    @functools.partial(
        pl.kernel,
        out_type=jax.ShapeDtypeStruct((NCORES * n_pad, d), F32),
        mesh=plsc.VectorSubcoreMesh(core_axis_name="c", subcore_axis_name="s"),
        scratch_types=[
            pltpu.VMEM((grp, CH), jnp.int32),
            pltpu.VMEM((grp, CH), jnp.int32),
            pltpu.VMEM((2, CH, d), F32),
            pltpu.VMEM_SHARED((n_pad, d), F32),
            pltpu.SemaphoreType.DMA,
        ],
    )
    def sc_scatter(xs_hbm, row_hbm, col_hbm, out_hbm,
                   idx_row, idx_col, rows_v, agg_sh, gsem):
        c = lax.axis_index("c")
        s = lax.axis_index("s")
        tile = c * NSUB + s
        chunk0 = tile * cpt

        zv = jnp.zeros((16,), F32)

        def fill_zero(r, carry):
            for q in range(d // 16):
                rows_v[0, r, pl.ds(q * 16, 16)] = zv
            return carry

        lax.fori_loop(0, CH, fill_zero, 0)

        base = s * rpt
        for off in range(0, rpt - CH + 1, CH):
            pltpu.sync_copy(rows_v.at[0], agg_sh.at[pl.ds(base + off, CH)])
        if rpt % CH:
            pltpu.sync_copy(rows_v.at[0], agg_sh.at[pl.ds(base + rpt - CH, CH)])
        plsc.subcore_barrier()

        for g in range(2):
            pltpu.sync_copy(
                row_hbm.at[pl.ds(chunk0 + g * grp, grp)], idx_row)
            pltpu.sync_copy(
                col_hbm.at[pl.ds(chunk0 + g * grp, grp)], idx_col)

            def chunk_body(j, carry):
                pltpu.async_copy(xs_hbm.at[idx_row.at[j]],
                                 rows_v.at[0], gsem).wait()
                pltpu.sync_copy(rows_v.at[0],
                                agg_sh.at[idx_col.at[j]], add=True)
                return carry

            lax.fori_loop(0, grp, chunk_body, 0)

        plsc.subcore_barrier()
        pltpu.sync_copy(agg_sh.at[pl.ds(base, rpt)],
                        out_hbm.at[pl.ds(c * n_pad + base, rpt)])

    return sc_scatter


def _tc_pre_body(x_ref, win_ref, bin_ref, degp_ref, w0_ref,
                 h_ref, xs_ref, dis_ref):
    n = h_ref.shape[0]
    n_pad = degp_ref.shape[0] // 2
    deg = degp_ref[:n, 0:1] + degp_ref[n_pad:n_pad + n, 0:1] + 1.0
    dis = lax.rsqrt(deg)
    h = jnp.dot(x_ref[...], win_ref[...], preferred_element_type=F32,
                precision=lax.Precision.HIGHEST) + bin_ref[...]
    h_ref[...] = h
    xs_ref[...] = jnp.dot(h, w0_ref[...], preferred_element_type=F32,
                          precision=lax.Precision.HIGHEST) * dis
    dis_ref[...] = dis


def _combined_s(sp_ref, n):
    n_pad = sp_ref.shape[0] // 2
    return sp_ref[:n, :] + sp_ref[n_pad:n_pad + n, :]


def _tc_bn_body(sp_ref, xs_ref, h_ref, dis_ref, bcv_ref, gam_ref, bet_ref,
                hout_ref):
    n = h_ref.shape[0]
    dis = dis_ref[...]
    agg = dis * (_combined_s(sp_ref, n) + xs_ref[...]) + bcv_ref[...]
    mean = jnp.mean(agg, axis=0, keepdims=True)
    ctr = agg - mean
    var = jnp.mean(ctr * ctr, axis=0, keepdims=True)
    hn = gam_ref[...] * ctr * lax.rsqrt(var + EPS) + bet_ref[...]
    hn = jnp.maximum(hn, 0.0)
    hout_ref[...] = h_ref[...] + hn


def _tc_mm_body(h_ref, dis_ref, wn_ref, xsn_ref):
    xsn_ref[...] = jnp.dot(h_ref[...], wn_ref[...], preferred_element_type=F32,
                           precision=lax.Precision.HIGHEST) * dis_ref[...]


def _tc_mmf_body(h_ref, wout_ref, bout_ref, y_ref):
    y_ref[...] = jnp.dot(h_ref[...], wout_ref[...], preferred_element_type=F32,
                         precision=lax.Precision.HIGHEST) + bout_ref[...]


def kernel(x, edge_index, W_in, b_in, W_conv, b_conv, gamma, beta,
           W_out, b_out):
    n, _ = x.shape
    d = W_in.shape[1]
    num_layers = W_conv.shape[0]
    e = edge_index.shape[1]

    n_pad = ((n + 64 + 63) // 64) * 64
    cpt = ((-(-e // (CH * NTILES)) + 15) // 16) * 16
    e_pad = cpt * CH * NTILES
    ncht = e_pad // CH

    pad = e_pad - e
    prow = (jnp.arange(pad, dtype=jnp.int32) * 97) % n
    pcol = n + (jnp.arange(pad, dtype=jnp.int32) % (n_pad - n))
    row_p = jnp.concatenate([edge_index[0], prow]).reshape(ncht, CH)
    col_p = jnp.concatenate([edge_index[1], pcol]).reshape(ncht, CH)

    sc_scatter = _build_sc_scatter(n_pad, d, ncht, cpt)

    degp = sc_scatter(jnp.ones((n, d), F32), row_p, col_p)

    b_in2 = b_in.reshape(1, d)
    b_out2 = b_out.reshape(1, d)

    h, xs, dis = pl.pallas_call(
        _tc_pre_body,
        out_shape=[
            jax.ShapeDtypeStruct((n, d), F32),
            jax.ShapeDtypeStruct((n, d), F32),
            jax.ShapeDtypeStruct((n, 1), F32),
        ],
    )(x, W_in, b_in2, degp, W_conv[0])

    for i in range(num_layers):
        sp = sc_scatter(xs, row_p, col_p)
        bcv = b_conv[i].reshape(1, d)
        gam = gamma[i].reshape(1, d)
        bet = beta[i].reshape(1, d)
        h = pl.pallas_call(
            _tc_bn_body,
            out_shape=jax.ShapeDtypeStruct((n, d), F32),
        )(sp, xs, h, dis, bcv, gam, bet)
        if i < num_layers - 1:
            xs = pl.pallas_call(
                _tc_mm_body,
                out_shape=jax.ShapeDtypeStruct((n, d), F32),
            )(h, dis, W_conv[i + 1])
        else:
            out = pl.pallas_call(
                _tc_mmf_body,
                out_shape=jax.ShapeDtypeStruct((n, d), F32),
            )(h, W_out, b_out2)
    return out


# speedup vs baseline: 17.2283x; 1.2850x over previous
import functools

import jax
import jax.numpy as jnp
from jax import lax
from jax.experimental import pallas as pl
from jax.experimental.pallas import tpu as pltpu
from jax.experimental.pallas import tpu_sc as plsc

EPS = 1e-5
F32 = jnp.float32
CH = 128
NCORES = 2
NSUB = 16
NTILES = NCORES * NSUB


def _build_sc_scatter(n_pad, d, ncht, cpt):
    rpt = n_pad // NSUB
    grp = cpt // 2

    @functools.partial(
        pl.kernel,
        out_type=jax.ShapeDtypeStruct((NCORES * n_pad, d), F32),
        mesh=plsc.VectorSubcoreMesh(core_axis_name="c", subcore_axis_name="s"),
        scratch_types=[
            pltpu.VMEM((grp, CH), jnp.int32),
            pltpu.VMEM((grp, CH), jnp.int32),
            pltpu.VMEM((2, CH, d), F32),
            pltpu.VMEM_SHARED((n_pad, d), F32),
            pltpu.SemaphoreType.DMA,
            pltpu.SemaphoreType.DMA,
        ],
    )
    def sc_scatter(xs_hbm, row_hbm, col_hbm, out_hbm,
                   idx_row, idx_col, rows_v, agg_sh, gsem, ssem):
        c = lax.axis_index("c")
        s = lax.axis_index("s")
        tile = c * NSUB + s
        chunk0 = tile * cpt

        zv = jnp.zeros((16,), F32)

        def fill_zero(r, carry):
            for q in range(d // 16):
                rows_v[0, r, pl.ds(q * 16, 16)] = zv
            return carry

        lax.fori_loop(0, CH, fill_zero, 0)

        base = s * rpt
        for off in range(0, rpt - CH + 1, CH):
            pltpu.sync_copy(rows_v.at[0], agg_sh.at[pl.ds(base + off, CH)])
        if rpt % CH:
            pltpu.sync_copy(rows_v.at[0], agg_sh.at[pl.ds(base + rpt - CH, CH)])
        plsc.subcore_barrier()

        def g_start(j, b):
            pltpu.async_copy(xs_hbm.at[idx_row.at[j]], rows_v.at[b], gsem)

        def g_wait():
            pltpu.make_async_copy(xs_hbm.at[idx_row.at[0]],
                                  rows_v.at[0], gsem).wait()

        def s_start(j, b):
            pltpu.async_copy(rows_v.at[b], agg_sh.at[idx_col.at[j]],
                             ssem, add=True)

        def s_wait():
            pltpu.make_async_copy(rows_v.at[0],
                                  agg_sh.at[idx_col.at[0]], ssem).wait()

        for g in range(2):
            pltpu.sync_copy(
                row_hbm.at[pl.ds(chunk0 + g * grp, grp)], idx_row)
            pltpu.sync_copy(
                col_hbm.at[pl.ds(chunk0 + g * grp, grp)], idx_col)

            g_start(0, 0)

            def chunk_body(j, carry):
                b = lax.rem(j, 2)
                g_wait()

                @pl.when(j > 0)
                def _():
                    s_wait()

                @pl.when(j + 1 < grp)
                def _():
                    g_start(j + 1, 1 - b)

                s_start(j, b)
                return carry

            lax.fori_loop(0, grp, chunk_body, 0)
            s_wait()

        plsc.subcore_barrier()
        pltpu.sync_copy(agg_sh.at[pl.ds(base, rpt)],
                        out_hbm.at[pl.ds(c * n_pad + base, rpt)])

    return sc_scatter


def _tc_pre_body(x_ref, win_ref, bin_ref, degp_ref, w0_ref,
                 h_ref, xs_ref, dis_ref):
    n = h_ref.shape[0]
    n_pad = degp_ref.shape[0] // 2
    deg = degp_ref[:n, 0:1] + degp_ref[n_pad:n_pad + n, 0:1] + 1.0
    dis = lax.rsqrt(deg)
    h = jnp.dot(x_ref[...], win_ref[...], preferred_element_type=F32,
                precision=lax.Precision.HIGHEST) + bin_ref[...]
    h_ref[...] = h
    xs_ref[...] = jnp.dot(h, w0_ref[...], preferred_element_type=F32,
                          precision=lax.Precision.HIGHEST) * dis
    dis_ref[...] = dis


def _combined_s(sp_ref, n):
    n_pad = sp_ref.shape[0] // 2
    return sp_ref[:n, :] + sp_ref[n_pad:n_pad + n, :]


def _tc_bn_body(sp_ref, xs_ref, h_ref, dis_ref, bcv_ref, gam_ref, bet_ref,
                hout_ref):
    n = h_ref.shape[0]
    dis = dis_ref[...]
    agg = dis * (_combined_s(sp_ref, n) + xs_ref[...]) + bcv_ref[...]
    mean = jnp.mean(agg, axis=0, keepdims=True)
    ctr = agg - mean
    var = jnp.mean(ctr * ctr, axis=0, keepdims=True)
    hn = gam_ref[...] * ctr * lax.rsqrt(var + EPS) + bet_ref[...]
    hn = jnp.maximum(hn, 0.0)
    hout_ref[...] = h_ref[...] + hn


def _tc_mm_body(h_ref, dis_ref, wn_ref, xsn_ref):
    xsn_ref[...] = jnp.dot(h_ref[...], wn_ref[...], preferred_element_type=F32,
                           precision=lax.Precision.HIGHEST) * dis_ref[...]


def _tc_mmf_body(h_ref, wout_ref, bout_ref, y_ref):
    y_ref[...] = jnp.dot(h_ref[...], wout_ref[...], preferred_element_type=F32,
                         precision=lax.Precision.HIGHEST) + bout_ref[...]


def kernel(x, edge_index, W_in, b_in, W_conv, b_conv, gamma, beta,
           W_out, b_out):
    n, _ = x.shape
    d = W_in.shape[1]
    num_layers = W_conv.shape[0]
    e = edge_index.shape[1]

    n_pad = ((n + 64 + 63) // 64) * 64
    cpt = ((-(-e // (CH * NTILES)) + 15) // 16) * 16
    e_pad = cpt * CH * NTILES
    ncht = e_pad // CH

    pad = e_pad - e
    prow = (jnp.arange(pad, dtype=jnp.int32) * 97) % n
    pcol = n + (jnp.arange(pad, dtype=jnp.int32) % (n_pad - n))
    row_p = jnp.concatenate([edge_index[0], prow]).reshape(ncht, CH)
    col_p = jnp.concatenate([edge_index[1], pcol]).reshape(ncht, CH)

    sc_scatter = _build_sc_scatter(n_pad, d, ncht, cpt)

    degp = sc_scatter(jnp.ones((n, d), F32), row_p, col_p)

    b_in2 = b_in.reshape(1, d)
    b_out2 = b_out.reshape(1, d)

    h, xs, dis = pl.pallas_call(
        _tc_pre_body,
        out_shape=[
            jax.ShapeDtypeStruct((n, d), F32),
            jax.ShapeDtypeStruct((n, d), F32),
            jax.ShapeDtypeStruct((n, 1), F32),
        ],
    )(x, W_in, b_in2, degp, W_conv[0])

    for i in range(num_layers):
        sp = sc_scatter(xs, row_p, col_p)
        bcv = b_conv[i].reshape(1, d)
        gam = gamma[i].reshape(1, d)
        bet = beta[i].reshape(1, d)
        h = pl.pallas_call(
            _tc_bn_body,
            out_shape=jax.ShapeDtypeStruct((n, d), F32),
        )(sp, xs, h, dis, bcv, gam, bet)
        if i < num_layers - 1:
            xs = pl.pallas_call(
                _tc_mm_body,
                out_shape=jax.ShapeDtypeStruct((n, d), F32),
            )(h, dis, W_conv[i + 1])
        else:
            out = pl.pallas_call(
                _tc_mmf_body,
                out_shape=jax.ShapeDtypeStruct((n, d), F32),
            )(h, W_out, b_out2)
    return out


# speedup vs baseline: 19.1304x; 1.1104x over previous
import functools

import jax
import jax.numpy as jnp
from jax import lax
from jax.experimental import pallas as pl
from jax.experimental.pallas import tpu as pltpu
from jax.experimental.pallas import tpu_sc as plsc

EPS = 1e-5
F32 = jnp.float32
CH = 128
NCORES = 2
NSUB = 16
NTILES = NCORES * NSUB


def _build_sc_ones_scatter(n_pad, d, ncht, cpt):
    rpt = n_pad // NSUB
    grp = cpt // 2

    @functools.partial(
        pl.kernel,
        out_type=jax.ShapeDtypeStruct((NCORES * n_pad, d), F32),
        mesh=plsc.VectorSubcoreMesh(core_axis_name="c", subcore_axis_name="s"),
        scratch_types=[
            pltpu.VMEM((grp, CH), jnp.int32),
            pltpu.VMEM((CH, d), F32),
            pltpu.VMEM_SHARED((n_pad, d), F32),
            pltpu.SemaphoreType.DMA,
        ],
    )
    def sc_ones(col_hbm, out_hbm, idx_col, ones_v, agg_sh, ssem):
        c = lax.axis_index("c")
        s = lax.axis_index("s")
        tile = c * NSUB + s
        chunk0 = tile * cpt

        def fill(val):
            vv = jnp.full((16,), val, F32)

            def body(r, carry):
                for q in range(d // 16):
                    ones_v[r, pl.ds(q * 16, 16)] = vv
                return carry

            lax.fori_loop(0, CH, body, 0)

        fill(0.0)
        base = s * rpt
        for off in range(0, rpt - CH + 1, CH):
            pltpu.sync_copy(ones_v, agg_sh.at[pl.ds(base + off, CH)])
        if rpt % CH:
            pltpu.sync_copy(ones_v, agg_sh.at[pl.ds(base + rpt - CH, CH)])
        fill(1.0)
        plsc.subcore_barrier()

        def s_start(j):
            pltpu.async_copy(ones_v, agg_sh.at[idx_col.at[j]],
                             ssem, add=True)

        def s_wait():
            pltpu.make_async_copy(ones_v,
                                  agg_sh.at[idx_col.at[0]], ssem).wait()

        for g in range(2):
            pltpu.sync_copy(
                col_hbm.at[pl.ds(chunk0 + g * grp, grp)], idx_col)
            s_start(0)
            s_start(1)

            def chunk_body(j, carry):
                s_start(j + 2)
                s_wait()
                return carry

            lax.fori_loop(0, grp - 2, chunk_body, 0)
            s_wait()
            s_wait()

        plsc.subcore_barrier()
        pltpu.sync_copy(agg_sh.at[pl.ds(base, rpt)],
                        out_hbm.at[pl.ds(c * n_pad + base, rpt)])

    return sc_ones


def _build_sc_scatter(n_pad, d, ncht, cpt):
    rpt = n_pad // NSUB
    grp = cpt // 2

    @functools.partial(
        pl.kernel,
        out_type=jax.ShapeDtypeStruct((NCORES * n_pad, d), F32),
        mesh=plsc.VectorSubcoreMesh(core_axis_name="c", subcore_axis_name="s"),
        scratch_types=[
            pltpu.VMEM((grp, CH), jnp.int32),
            pltpu.VMEM((grp, CH), jnp.int32),
            pltpu.VMEM((2, CH, d), F32),
            pltpu.VMEM_SHARED((n_pad, d), F32),
            pltpu.SemaphoreType.DMA,
            pltpu.SemaphoreType.DMA,
        ],
    )
    def sc_scatter(xs_hbm, row_hbm, col_hbm, out_hbm,
                   idx_row, idx_col, rows_v, agg_sh, gsem, ssem):
        c = lax.axis_index("c")
        s = lax.axis_index("s")
        tile = c * NSUB + s
        chunk0 = tile * cpt

        zv = jnp.zeros((16,), F32)

        def fill_zero(r, carry):
            for q in range(d // 16):
                rows_v[0, r, pl.ds(q * 16, 16)] = zv
            return carry

        lax.fori_loop(0, CH, fill_zero, 0)

        base = s * rpt
        for off in range(0, rpt - CH + 1, CH):
            pltpu.sync_copy(rows_v.at[0], agg_sh.at[pl.ds(base + off, CH)])
        if rpt % CH:
            pltpu.sync_copy(rows_v.at[0], agg_sh.at[pl.ds(base + rpt - CH, CH)])
        plsc.subcore_barrier()

        def g_start(j, b):
            pltpu.async_copy(xs_hbm.at[idx_row.at[j]], rows_v.at[b], gsem)

        def g_wait():
            pltpu.make_async_copy(xs_hbm.at[idx_row.at[0]],
                                  rows_v.at[0], gsem).wait()

        def s_start(j, b):
            pltpu.async_copy(rows_v.at[b], agg_sh.at[idx_col.at[j]],
                             ssem, add=True)

        def s_wait():
            pltpu.make_async_copy(rows_v.at[0],
                                  agg_sh.at[idx_col.at[0]], ssem).wait()

        for g in range(2):
            pltpu.sync_copy(
                row_hbm.at[pl.ds(chunk0 + g * grp, grp)], idx_row)
            pltpu.sync_copy(
                col_hbm.at[pl.ds(chunk0 + g * grp, grp)], idx_col)

            g_start(0, 0)

            def chunk_body(j, carry):
                b = lax.rem(j, 2)
                g_wait()

                @pl.when(j > 0)
                def _():
                    s_wait()

                @pl.when(j + 1 < grp)
                def _():
                    g_start(j + 1, 1 - b)

                s_start(j, b)
                return carry

            lax.fori_loop(0, grp, chunk_body, 0)
            s_wait()

        plsc.subcore_barrier()
        pltpu.sync_copy(agg_sh.at[pl.ds(base, rpt)],
                        out_hbm.at[pl.ds(c * n_pad + base, rpt)])

    return sc_scatter


def _tc_pre_body(x_ref, win_ref, bin_ref, degp_ref, w0_ref,
                 h_ref, xs_ref, dis_ref):
    n = h_ref.shape[0]
    n_pad = degp_ref.shape[0] // 2
    deg = degp_ref[:n, 0:1] + degp_ref[n_pad:n_pad + n, 0:1] + 1.0
    dis = lax.rsqrt(deg)
    h = jnp.dot(x_ref[...], win_ref[...], preferred_element_type=F32,
                precision=lax.Precision.HIGHEST) + bin_ref[...]
    h_ref[...] = h
    xs_ref[...] = jnp.dot(h, w0_ref[...], preferred_element_type=F32,
                          precision=lax.Precision.HIGHEST) * dis
    dis_ref[...] = dis


def _combined_s(sp_ref, n):
    n_pad = sp_ref.shape[0] // 2
    return sp_ref[:n, :] + sp_ref[n_pad:n_pad + n, :]


def _bn_relu_res(sp_ref, xs_ref, h_ref, dis, bcv_ref, gam_ref, bet_ref):
    n = h_ref.shape[0]
    agg = dis * (_combined_s(sp_ref, n) + xs_ref[...]) + bcv_ref[...]
    mean = jnp.mean(agg, axis=0, keepdims=True)
    ctr = agg - mean
    var = jnp.mean(ctr * ctr, axis=0, keepdims=True)
    hn = gam_ref[...] * ctr * lax.rsqrt(var + EPS) + bet_ref[...]
    hn = jnp.maximum(hn, 0.0)
    return h_ref[...] + hn


def _tc_mid_body(sp_ref, xs_ref, h_ref, dis_ref, bcv_ref, gam_ref, bet_ref,
                 wn_ref, hout_ref, xsn_ref):
    dis = dis_ref[...]
    h_new = _bn_relu_res(sp_ref, xs_ref, h_ref, dis, bcv_ref, gam_ref,
                         bet_ref)
    hout_ref[...] = h_new
    xsn_ref[...] = jnp.dot(h_new, wn_ref[...], preferred_element_type=F32,
                           precision=lax.Precision.HIGHEST) * dis


def _tc_final_body(sp_ref, xs_ref, h_ref, dis_ref, bcv_ref, gam_ref, bet_ref,
                   wout_ref, bout_ref, y_ref):
    h_new = _bn_relu_res(sp_ref, xs_ref, h_ref, dis_ref[...], bcv_ref,
                         gam_ref, bet_ref)
    y_ref[...] = jnp.dot(h_new, wout_ref[...], preferred_element_type=F32,
                         precision=lax.Precision.HIGHEST) + bout_ref[...]


def kernel(x, edge_index, W_in, b_in, W_conv, b_conv, gamma, beta,
           W_out, b_out):
    n, _ = x.shape
    d = W_in.shape[1]
    num_layers = W_conv.shape[0]
    e = edge_index.shape[1]

    n_pad = ((n + 64 + 63) // 64) * 64
    cpt = ((-(-e // (CH * NTILES)) + 15) // 16) * 16
    e_pad = cpt * CH * NTILES
    ncht = e_pad // CH

    pad = e_pad - e
    prow = (jnp.arange(pad, dtype=jnp.int32) * 97) % n
    pcol = n + (jnp.arange(pad, dtype=jnp.int32) % (n_pad - n))
    row_p = jnp.concatenate([edge_index[0], prow]).reshape(ncht, CH)
    col_p = jnp.concatenate([edge_index[1], pcol]).reshape(ncht, CH)

    sc_ones = _build_sc_ones_scatter(n_pad, d, ncht, cpt)
    sc_scatter = _build_sc_scatter(n_pad, d, ncht, cpt)

    degp = sc_ones(col_p)

    b_in2 = b_in.reshape(1, d)
    b_out2 = b_out.reshape(1, d)

    h, xs, dis = pl.pallas_call(
        _tc_pre_body,
        out_shape=[
            jax.ShapeDtypeStruct((n, d), F32),
            jax.ShapeDtypeStruct((n, d), F32),
            jax.ShapeDtypeStruct((n, 1), F32),
        ],
    )(x, W_in, b_in2, degp, W_conv[0])

    for i in range(num_layers):
        sp = sc_scatter(xs, row_p, col_p)
        bcv = b_conv[i].reshape(1, d)
        gam = gamma[i].reshape(1, d)
        bet = beta[i].reshape(1, d)
        cp = pltpu.CompilerParams(vmem_limit_bytes=64 * 1024 * 1024)
        if i < num_layers - 1:
            h, xs = pl.pallas_call(
                _tc_mid_body,
                out_shape=[
                    jax.ShapeDtypeStruct((n, d), F32),
                    jax.ShapeDtypeStruct((n, d), F32),
                ],
                compiler_params=cp,
            )(sp, xs, h, dis, bcv, gam, bet, W_conv[i + 1])
        else:
            out = pl.pallas_call(
                _tc_final_body,
                out_shape=jax.ShapeDtypeStruct((n, d), F32),
                compiler_params=cp,
            )(sp, xs, h, dis, bcv, gam, bet, W_out, b_out2)
    return out


# speedup vs baseline: 22.0238x; 1.1512x over previous
import functools

import jax
import jax.numpy as jnp
from jax import lax
from jax.experimental import pallas as pl
from jax.experimental.pallas import tpu as pltpu
from jax.experimental.pallas import tpu_sc as plsc

EPS = 1e-5
F32 = jnp.float32
CH = 128
NCORES = 2
NSUB = 16
NTILES = NCORES * NSUB


def _build_sc_ones_scatter(n_pad, d, ncht, cpt):
    rpt = n_pad // NSUB
    grp = cpt // 2

    @functools.partial(
        pl.kernel,
        out_type=jax.ShapeDtypeStruct((NCORES * n_pad, d), F32),
        mesh=plsc.VectorSubcoreMesh(core_axis_name="c", subcore_axis_name="s"),
        scratch_types=[
            pltpu.VMEM((grp, CH), jnp.int32),
            pltpu.VMEM((CH, d), F32),
            pltpu.VMEM_SHARED((n_pad, d), F32),
            pltpu.SemaphoreType.DMA,
        ],
    )
    def sc_ones(col_hbm, out_hbm, idx_col, ones_v, agg_sh, ssem):
        c = lax.axis_index("c")
        s = lax.axis_index("s")
        tile = c * NSUB + s
        chunk0 = tile * cpt

        def fill(val):
            vv = jnp.full((16,), val, F32)

            def body(r, carry):
                for q in range(d // 16):
                    ones_v[r, pl.ds(q * 16, 16)] = vv
                return carry

            lax.fori_loop(0, CH, body, 0)

        fill(0.0)
        base = s * rpt
        for off in range(0, rpt - CH + 1, CH):
            pltpu.sync_copy(ones_v, agg_sh.at[pl.ds(base + off, CH)])
        if rpt % CH:
            pltpu.sync_copy(ones_v, agg_sh.at[pl.ds(base + rpt - CH, CH)])
        fill(1.0)
        plsc.subcore_barrier()

        def s_start(j):
            pltpu.async_copy(ones_v, agg_sh.at[idx_col.at[j]],
                             ssem, add=True)

        def s_wait():
            pltpu.make_async_copy(ones_v,
                                  agg_sh.at[idx_col.at[0]], ssem).wait()

        for g in range(2):
            pltpu.sync_copy(
                col_hbm.at[pl.ds(chunk0 + g * grp, grp)], idx_col)
            s_start(0)
            s_start(1)

            def chunk_body(j, carry):
                s_start(j + 2)
                s_wait()
                return carry

            lax.fori_loop(0, grp - 2, chunk_body, 0)
            s_wait()
            s_wait()

        plsc.subcore_barrier()
        pltpu.sync_copy(agg_sh.at[pl.ds(base, rpt)],
                        out_hbm.at[pl.ds(c * n_pad + base, rpt)])

    return sc_ones


def _build_sc_scatter(n_pad, d, ncht, cpt):
    rpt = n_pad // NSUB
    grp = cpt // 2

    @functools.partial(
        pl.kernel,
        out_type=jax.ShapeDtypeStruct((NCORES * n_pad, d), F32),
        mesh=plsc.VectorSubcoreMesh(core_axis_name="c", subcore_axis_name="s"),
        scratch_types=[
            pltpu.VMEM((grp, CH), jnp.int32),
            pltpu.VMEM((grp, CH), jnp.int32),
            pltpu.VMEM((2, CH, d), F32),
            pltpu.VMEM_SHARED((n_pad, d), F32),
            pltpu.SemaphoreType.DMA,
            pltpu.SemaphoreType.DMA,
        ],
    )
    def sc_scatter(xs_hbm, row_hbm, col_hbm, out_hbm,
                   idx_row, idx_col, rows_v, agg_sh, gsem, ssem):
        c = lax.axis_index("c")
        s = lax.axis_index("s")
        tile = c * NSUB + s
        chunk0 = tile * cpt

        zv = jnp.zeros((16,), F32)

        def fill_zero(r, carry):
            for q in range(d // 16):
                rows_v[0, r, pl.ds(q * 16, 16)] = zv
            return carry

        lax.fori_loop(0, CH, fill_zero, 0)

        base = s * rpt
        for off in range(0, rpt - CH + 1, CH):
            pltpu.sync_copy(rows_v.at[0], agg_sh.at[pl.ds(base + off, CH)])
        if rpt % CH:
            pltpu.sync_copy(rows_v.at[0], agg_sh.at[pl.ds(base + rpt - CH, CH)])
        plsc.subcore_barrier()

        def g_start(j, b):
            pltpu.async_copy(xs_hbm.at[idx_row.at[j]], rows_v.at[b], gsem)

        def g_wait():
            pltpu.make_async_copy(xs_hbm.at[idx_row.at[0]],
                                  rows_v.at[0], gsem).wait()

        def s_start(j, b):
            pltpu.async_copy(rows_v.at[b], agg_sh.at[idx_col.at[j]],
                             ssem, add=True)

        def s_wait():
            pltpu.make_async_copy(rows_v.at[0],
                                  agg_sh.at[idx_col.at[0]], ssem).wait()

        for g in range(2):
            pltpu.sync_copy(
                row_hbm.at[pl.ds(chunk0 + g * grp, grp)], idx_row)
            pltpu.sync_copy(
                col_hbm.at[pl.ds(chunk0 + g * grp, grp)], idx_col)

            g_start(0, 0)

            def chunk_body(j, carry):
                b = lax.rem(j, 2)

                @pl.when(j > 0)
                def _():
                    s_wait()

                @pl.when(j + 1 < grp)
                def _():
                    g_start(j + 1, 1 - b)

                g_wait()
                s_start(j, b)
                return carry

            lax.fori_loop(0, grp, chunk_body, 0)
            s_wait()

        plsc.subcore_barrier()
        pltpu.sync_copy(agg_sh.at[pl.ds(base, rpt)],
                        out_hbm.at[pl.ds(c * n_pad + base, rpt)])

    return sc_scatter


def _tc_pre_body(x_ref, win_ref, bin_ref, degp_ref, w0_ref,
                 h_ref, xs_ref, dis_ref):
    n = h_ref.shape[0]
    n_pad = degp_ref.shape[0] // 2
    deg = degp_ref[:n, 0:1] + degp_ref[n_pad:n_pad + n, 0:1] + 1.0
    dis = lax.rsqrt(deg)
    h = jnp.dot(x_ref[...], win_ref[...], preferred_element_type=F32,
                precision=lax.Precision.HIGHEST) + bin_ref[...]
    h_ref[...] = h
    xs_ref[...] = jnp.dot(h, w0_ref[...], preferred_element_type=F32,
                          precision=lax.Precision.HIGHEST) * dis
    dis_ref[...] = dis


def _combined_s(sp_ref, n):
    n_pad = sp_ref.shape[0] // 2
    return sp_ref[:n, :] + sp_ref[n_pad:n_pad + n, :]


def _bn_relu_res(sp_ref, xs_ref, h_ref, dis, bcv_ref, gam_ref, bet_ref):
    n = h_ref.shape[0]
    agg = dis * (_combined_s(sp_ref, n) + xs_ref[...]) + bcv_ref[...]
    mean = jnp.mean(agg, axis=0, keepdims=True)
    ctr = agg - mean
    var = jnp.mean(ctr * ctr, axis=0, keepdims=True)
    hn = gam_ref[...] * ctr * lax.rsqrt(var + EPS) + bet_ref[...]
    hn = jnp.maximum(hn, 0.0)
    return h_ref[...] + hn


def _tc_mid_body(sp_ref, xs_ref, h_ref, dis_ref, bcv_ref, gam_ref, bet_ref,
                 wn_ref, hout_ref, xsn_ref):
    dis = dis_ref[...]
    h_new = _bn_relu_res(sp_ref, xs_ref, h_ref, dis, bcv_ref, gam_ref,
                         bet_ref)
    hout_ref[...] = h_new
    xsn_ref[...] = jnp.dot(h_new, wn_ref[...], preferred_element_type=F32,
                           precision=lax.Precision.HIGHEST) * dis


def _tc_final_body(sp_ref, xs_ref, h_ref, dis_ref, bcv_ref, gam_ref, bet_ref,
                   wout_ref, bout_ref, y_ref):
    h_new = _bn_relu_res(sp_ref, xs_ref, h_ref, dis_ref[...], bcv_ref,
                         gam_ref, bet_ref)
    y_ref[...] = jnp.dot(h_new, wout_ref[...], preferred_element_type=F32,
                         precision=lax.Precision.HIGHEST) + bout_ref[...]


def kernel(x, edge_index, W_in, b_in, W_conv, b_conv, gamma, beta,
           W_out, b_out):
    n, _ = x.shape
    d = W_in.shape[1]
    num_layers = W_conv.shape[0]
    e = edge_index.shape[1]

    n_pad = ((n + 64 + 63) // 64) * 64
    cpt = ((-(-e // (CH * NTILES)) + 15) // 16) * 16
    e_pad = cpt * CH * NTILES
    ncht = e_pad // CH

    pad = e_pad - e
    prow = (jnp.arange(pad, dtype=jnp.int32) * 97) % n
    pcol = n + (jnp.arange(pad, dtype=jnp.int32) % (n_pad - n))
    row_p = jnp.concatenate([edge_index[0], prow]).reshape(ncht, CH)
    col_p = jnp.concatenate([edge_index[1], pcol]).reshape(ncht, CH)

    sc_ones = _build_sc_ones_scatter(n_pad, d, ncht, cpt)
    sc_scatter = _build_sc_scatter(n_pad, d, ncht, cpt)

    degp = sc_ones(col_p)

    b_in2 = b_in.reshape(1, d)
    b_out2 = b_out.reshape(1, d)

    h, xs, dis = pl.pallas_call(
        _tc_pre_body,
        out_shape=[
            jax.ShapeDtypeStruct((n, d), F32),
            jax.ShapeDtypeStruct((n, d), F32),
            jax.ShapeDtypeStruct((n, 1), F32),
        ],
    )(x, W_in, b_in2, degp, W_conv[0])

    for i in range(num_layers):
        sp = sc_scatter(xs, row_p, col_p)
        bcv = b_conv[i].reshape(1, d)
        gam = gamma[i].reshape(1, d)
        bet = beta[i].reshape(1, d)
        cp = pltpu.CompilerParams(vmem_limit_bytes=64 * 1024 * 1024)
        if i < num_layers - 1:
            h, xs = pl.pallas_call(
                _tc_mid_body,
                out_shape=[
                    jax.ShapeDtypeStruct((n, d), F32),
                    jax.ShapeDtypeStruct((n, d), F32),
                ],
                compiler_params=cp,
            )(sp, xs, h, dis, bcv, gam, bet, W_conv[i + 1])
        else:
            out = pl.pallas_call(
                _tc_final_body,
                out_shape=jax.ShapeDtypeStruct((n, d), F32),
                compiler_params=cp,
            )(sp, xs, h, dis, bcv, gam, bet, W_out, b_out2)
    return out


# speedup vs baseline: 22.9073x; 1.0401x over previous
import functools

import jax
import jax.numpy as jnp
from jax import lax
from jax.experimental import pallas as pl
from jax.experimental.pallas import tpu as pltpu
from jax.experimental.pallas import tpu_sc as plsc

EPS = 1e-5
F32 = jnp.float32
CH = 128
NCORES = 2
NSUB = 16
NTILES = NCORES * NSUB


def _build_sc_ones_scatter(n_pad, d, ncht, cpt):
    rpt = n_pad // NSUB
    grp = cpt // 2

    @functools.partial(
        pl.kernel,
        out_type=jax.ShapeDtypeStruct((NCORES * n_pad, d), F32),
        mesh=plsc.VectorSubcoreMesh(core_axis_name="c", subcore_axis_name="s"),
        scratch_types=[
            pltpu.VMEM((grp, CH), jnp.int32),
            pltpu.VMEM((CH, d), F32),
            pltpu.VMEM_SHARED((n_pad, d), F32),
            pltpu.SemaphoreType.DMA,
        ],
    )
    def sc_ones(col_hbm, out_hbm, idx_col, ones_v, agg_sh, ssem):
        c = lax.axis_index("c")
        s = lax.axis_index("s")
        tile = c * NSUB + s
        chunk0 = tile * cpt

        def fill(val):
            vv = jnp.full((16,), val, F32)

            def body(r, carry):
                for q in range(d // 16):
                    ones_v[r, pl.ds(q * 16, 16)] = vv
                return carry

            lax.fori_loop(0, CH, body, 0)

        fill(0.0)
        base = s * rpt
        for off in range(0, rpt - CH + 1, CH):
            pltpu.sync_copy(ones_v, agg_sh.at[pl.ds(base + off, CH)])
        if rpt % CH:
            pltpu.sync_copy(ones_v, agg_sh.at[pl.ds(base + rpt - CH, CH)])
        fill(1.0)
        plsc.subcore_barrier()

        def s_start(j):
            pltpu.async_copy(ones_v, agg_sh.at[idx_col.at[j]],
                             ssem, add=True)

        def s_wait():
            pltpu.make_async_copy(ones_v,
                                  agg_sh.at[idx_col.at[0]], ssem).wait()

        for g in range(2):
            pltpu.sync_copy(
                col_hbm.at[pl.ds(chunk0 + g * grp, grp)], idx_col)
            s_start(0)
            s_start(1)

            def chunk_body(j, carry):
                s_start(j + 2)
                s_wait()
                return carry

            lax.fori_loop(0, grp - 2, chunk_body, 0)
            s_wait()
            s_wait()

        plsc.subcore_barrier()
        pltpu.sync_copy(agg_sh.at[pl.ds(base, rpt)],
                        out_hbm.at[pl.ds(c * n_pad + base, rpt)])

    return sc_ones


def _build_sc_scatter(n_pad, d, ncht, cpt):
    rpt = n_pad // NSUB
    grp = cpt // 2

    @functools.partial(
        pl.kernel,
        out_type=jax.ShapeDtypeStruct((NCORES * n_pad, d), F32),
        mesh=plsc.VectorSubcoreMesh(core_axis_name="c", subcore_axis_name="s"),
        scratch_types=[
            pltpu.VMEM((grp, CH), jnp.int32),
            pltpu.VMEM((grp, CH), jnp.int32),
            pltpu.VMEM((2, CH, d), F32),
            pltpu.VMEM_SHARED((n_pad, d), F32),
            pltpu.SemaphoreType.DMA,
            pltpu.SemaphoreType.DMA,
        ],
    )
    def sc_scatter(xs_hbm, row_hbm, col_hbm, out_hbm,
                   idx_row, idx_col, rows_v, agg_sh, gsem, ssem):
        c = lax.axis_index("c")
        s = lax.axis_index("s")
        tile = c * NSUB + s
        chunk0 = tile * cpt

        zv = jnp.zeros((16,), F32)

        def fill_zero(r, carry):
            for q in range(d // 16):
                rows_v[0, r, pl.ds(q * 16, 16)] = zv
            return carry

        lax.fori_loop(0, CH, fill_zero, 0)

        base = s * rpt
        for off in range(0, rpt - CH + 1, CH):
            pltpu.sync_copy(rows_v.at[0], agg_sh.at[pl.ds(base + off, CH)])
        if rpt % CH:
            pltpu.sync_copy(rows_v.at[0], agg_sh.at[pl.ds(base + rpt - CH, CH)])
        plsc.subcore_barrier()

        def g_start(j, b):
            pltpu.async_copy(xs_hbm.at[idx_row.at[j]], rows_v.at[b], gsem)

        def g_wait():
            pltpu.make_async_copy(xs_hbm.at[idx_row.at[0]],
                                  rows_v.at[0], gsem).wait()

        def s_start(j, b):
            pltpu.async_copy(rows_v.at[b], agg_sh.at[idx_col.at[j]],
                             ssem, add=True)

        def s_wait():
            pltpu.make_async_copy(rows_v.at[0],
                                  agg_sh.at[idx_col.at[0]], ssem).wait()

        for g in range(2):
            pltpu.sync_copy(
                row_hbm.at[pl.ds(chunk0 + g * grp, grp)], idx_row)
            pltpu.sync_copy(
                col_hbm.at[pl.ds(chunk0 + g * grp, grp)], idx_col)

            g_start(0, 0)

            def chunk_body(j, carry):
                b = lax.rem(j, 2)

                @pl.when(j > 0)
                def _():
                    s_wait()

                @pl.when(j + 1 < grp)
                def _():
                    g_start(j + 1, 1 - b)

                g_wait()
                s_start(j, b)
                return carry

            lax.fori_loop(0, grp, chunk_body, 0)
            s_wait()

        plsc.subcore_barrier()
        pltpu.sync_copy(agg_sh.at[pl.ds(base, rpt)],
                        out_hbm.at[pl.ds(c * n_pad + base, rpt)])

    return sc_scatter


def _tc_pre_body(x_ref, win_ref, bin_ref, degp_ref, w0_ref,
                 h_ref, xs_ref, dis_ref):
    n = h_ref.shape[0]
    n_pad = degp_ref.shape[0] // 2
    deg = degp_ref[:n, 0:1] + degp_ref[n_pad:n_pad + n, 0:1] + 1.0
    dis = lax.rsqrt(deg)
    h = jnp.dot(x_ref[...], win_ref[...], preferred_element_type=F32,
                precision=lax.Precision.DEFAULT) + bin_ref[...]
    h_ref[...] = h
    xs_ref[...] = jnp.dot(h, w0_ref[...], preferred_element_type=F32,
                          precision=lax.Precision.DEFAULT) * dis
    dis_ref[...] = dis


def _combined_s(sp_ref, n):
    n_pad = sp_ref.shape[0] // 2
    return sp_ref[:n, :] + sp_ref[n_pad:n_pad + n, :]


def _bn_relu_res(sp_ref, xs_ref, h_ref, dis, bcv_ref, gam_ref, bet_ref):
    n = h_ref.shape[0]
    agg = dis * (_combined_s(sp_ref, n) + xs_ref[...]) + bcv_ref[...]
    mean = jnp.mean(agg, axis=0, keepdims=True)
    ctr = agg - mean
    var = jnp.mean(ctr * ctr, axis=0, keepdims=True)
    hn = gam_ref[...] * ctr * lax.rsqrt(var + EPS) + bet_ref[...]
    hn = jnp.maximum(hn, 0.0)
    return h_ref[...] + hn


def _tc_mid_body(sp_ref, xs_ref, h_ref, dis_ref, bcv_ref, gam_ref, bet_ref,
                 wn_ref, hout_ref, xsn_ref):
    dis = dis_ref[...]
    h_new = _bn_relu_res(sp_ref, xs_ref, h_ref, dis, bcv_ref, gam_ref,
                         bet_ref)
    hout_ref[...] = h_new
    xsn_ref[...] = jnp.dot(h_new, wn_ref[...], preferred_element_type=F32,
                           precision=lax.Precision.DEFAULT) * dis


def _tc_final_body(sp_ref, xs_ref, h_ref, dis_ref, bcv_ref, gam_ref, bet_ref,
                   wout_ref, bout_ref, y_ref):
    h_new = _bn_relu_res(sp_ref, xs_ref, h_ref, dis_ref[...], bcv_ref,
                         gam_ref, bet_ref)
    y_ref[...] = jnp.dot(h_new, wout_ref[...], preferred_element_type=F32,
                         precision=lax.Precision.DEFAULT) + bout_ref[...]


def kernel(x, edge_index, W_in, b_in, W_conv, b_conv, gamma, beta,
           W_out, b_out):
    n, _ = x.shape
    d = W_in.shape[1]
    num_layers = W_conv.shape[0]
    e = edge_index.shape[1]

    n_pad = ((n + 64 + 63) // 64) * 64
    cpt = ((-(-e // (CH * NTILES)) + 15) // 16) * 16
    e_pad = cpt * CH * NTILES
    ncht = e_pad // CH

    pad = e_pad - e
    prow = (jnp.arange(pad, dtype=jnp.int32) * 97) % n
    pcol = n + (jnp.arange(pad, dtype=jnp.int32) % (n_pad - n))
    row_p = jnp.concatenate([edge_index[0], prow]).reshape(ncht, CH)
    col_p = jnp.concatenate([edge_index[1], pcol]).reshape(ncht, CH)

    sc_ones = _build_sc_ones_scatter(n_pad, d, ncht, cpt)
    sc_scatter = _build_sc_scatter(n_pad, d, ncht, cpt)

    degp = sc_ones(col_p)

    b_in2 = b_in.reshape(1, d)
    b_out2 = b_out.reshape(1, d)

    h, xs, dis = pl.pallas_call(
        _tc_pre_body,
        out_shape=[
            jax.ShapeDtypeStruct((n, d), F32),
            jax.ShapeDtypeStruct((n, d), F32),
            jax.ShapeDtypeStruct((n, 1), F32),
        ],
    )(x, W_in, b_in2, degp, W_conv[0])

    for i in range(num_layers):
        sp = sc_scatter(xs, row_p, col_p)
        bcv = b_conv[i].reshape(1, d)
        gam = gamma[i].reshape(1, d)
        bet = beta[i].reshape(1, d)
        cp = pltpu.CompilerParams(vmem_limit_bytes=64 * 1024 * 1024)
        if i < num_layers - 1:
            h, xs = pl.pallas_call(
                _tc_mid_body,
                out_shape=[
                    jax.ShapeDtypeStruct((n, d), F32),
                    jax.ShapeDtypeStruct((n, d), F32),
                ],
                compiler_params=cp,
            )(sp, xs, h, dis, bcv, gam, bet, W_conv[i + 1])
        else:
            out = pl.pallas_call(
                _tc_final_body,
                out_shape=jax.ShapeDtypeStruct((n, d), F32),
                compiler_params=cp,
            )(sp, xs, h, dis, bcv, gam, bet, W_out, b_out2)
    return out


# speedup vs baseline: 22.9309x; 1.0010x over previous
import functools

import jax
import jax.numpy as jnp
from jax import lax
from jax.experimental import pallas as pl
from jax.experimental.pallas import tpu as pltpu
from jax.experimental.pallas import tpu_sc as plsc

EPS = 1e-5
F32 = jnp.float32
CH = 128
NCORES = 2
NSUB = 16
NTILES = NCORES * NSUB


def _build_sc_ones_scatter(n_pad, d, ncht, cpt):
    rpt = n_pad // NSUB
    grp = cpt // 2

    @functools.partial(
        pl.kernel,
        out_type=jax.ShapeDtypeStruct((NCORES * n_pad, d), F32),
        mesh=plsc.VectorSubcoreMesh(core_axis_name="c", subcore_axis_name="s"),
        scratch_types=[
            pltpu.VMEM((grp, CH), jnp.int32),
            pltpu.VMEM((CH, d), F32),
            pltpu.VMEM_SHARED((n_pad, d), F32),
            pltpu.SemaphoreType.DMA,
        ],
    )
    def sc_ones(col_hbm, out_hbm, idx_col, ones_v, agg_sh, ssem):
        c = lax.axis_index("c")
        s = lax.axis_index("s")
        tile = c * NSUB + s
        chunk0 = tile * cpt

        def fill(val):
            vv = jnp.full((16,), val, F32)

            def body(r, carry):
                for q in range(d // 16):
                    ones_v[r, pl.ds(q * 16, 16)] = vv
                return carry

            lax.fori_loop(0, CH, body, 0)

        fill(0.0)
        base = s * rpt
        for off in range(0, rpt - CH + 1, CH):
            pltpu.sync_copy(ones_v, agg_sh.at[pl.ds(base + off, CH)])
        if rpt % CH:
            pltpu.sync_copy(ones_v, agg_sh.at[pl.ds(base + rpt - CH, CH)])
        fill(1.0)
        plsc.subcore_barrier()

        def s_start(j):
            pltpu.async_copy(ones_v, agg_sh.at[idx_col.at[j]],
                             ssem, add=True)

        def s_wait():
            pltpu.make_async_copy(ones_v,
                                  agg_sh.at[idx_col.at[0]], ssem).wait()

        for g in range(2):
            pltpu.sync_copy(
                col_hbm.at[pl.ds(chunk0 + g * grp, grp)], idx_col)
            s_start(0)
            s_start(1)

            def chunk_body(j, carry):
                s_start(j + 2)
                s_wait()
                return carry

            lax.fori_loop(0, grp - 2, chunk_body, 0)
            s_wait()
            s_wait()

        plsc.subcore_barrier()
        pltpu.sync_copy(agg_sh.at[pl.ds(base, rpt)],
                        out_hbm.at[pl.ds(c * n_pad + base, rpt)])

    return sc_ones


def _build_sc_scatter(n_pad, d, ncht, cpt):
    rpt = n_pad // NSUB
    grp = cpt // 2

    @functools.partial(
        pl.kernel,
        out_type=jax.ShapeDtypeStruct((NCORES * n_pad, d), F32),
        mesh=plsc.VectorSubcoreMesh(core_axis_name="c", subcore_axis_name="s"),
        scratch_types=[
            pltpu.VMEM((grp, CH), jnp.int32),
            pltpu.VMEM((grp, CH), jnp.int32),
            pltpu.VMEM((2, CH, d), F32),
            pltpu.VMEM_SHARED((n_pad, d), F32),
            pltpu.SemaphoreType.DMA,
            pltpu.SemaphoreType.DMA,
        ],
    )
    def sc_scatter(xs_hbm, row_hbm, col_hbm, out_hbm,
                   idx_row, idx_col, rows_v, agg_sh, gsem, ssem):
        c = lax.axis_index("c")
        s = lax.axis_index("s")
        tile = c * NSUB + s
        chunk0 = tile * cpt

        zv = jnp.zeros((16,), F32)

        def fill_zero(r, carry):
            for q in range(d // 16):
                rows_v[0, r, pl.ds(q * 16, 16)] = zv
            return carry

        lax.fori_loop(0, CH, fill_zero, 0)

        base = s * rpt
        for off in range(0, rpt - CH + 1, CH):
            pltpu.sync_copy(rows_v.at[0], agg_sh.at[pl.ds(base + off, CH)])
        if rpt % CH:
            pltpu.sync_copy(rows_v.at[0], agg_sh.at[pl.ds(base + rpt - CH, CH)])
        plsc.subcore_barrier()

        def g_start(j, b):
            pltpu.async_copy(xs_hbm.at[idx_row.at[j]], rows_v.at[b], gsem)

        def g_wait():
            pltpu.make_async_copy(xs_hbm.at[idx_row.at[0]],
                                  rows_v.at[0], gsem).wait()

        def s_start(j, b):
            pltpu.async_copy(rows_v.at[b], agg_sh.at[idx_col.at[j]],
                             ssem, add=True)

        def s_wait():
            pltpu.make_async_copy(rows_v.at[0],
                                  agg_sh.at[idx_col.at[0]], ssem).wait()

        for g in range(2):
            pltpu.sync_copy(
                row_hbm.at[pl.ds(chunk0 + g * grp, grp)], idx_row)
            pltpu.sync_copy(
                col_hbm.at[pl.ds(chunk0 + g * grp, grp)], idx_col)

            g_start(0, 0)

            def chunk_body(j, carry):
                b = lax.rem(j, 2)

                @pl.when(j > 0)
                def _():
                    s_wait()

                @pl.when(j + 1 < grp)
                def _():
                    g_start(j + 1, 1 - b)

                g_wait()
                s_start(j, b)
                return carry

            lax.fori_loop(0, grp, chunk_body, 0)
            s_wait()

        plsc.subcore_barrier()
        pltpu.sync_copy(agg_sh.at[pl.ds(base, rpt)],
                        out_hbm.at[pl.ds(c * n_pad + base, rpt)])

    return sc_scatter


def _tc_h0_body(x_ref, win_ref, bin_ref, h_ref):
    h_ref[...] = jnp.dot(x_ref[...], win_ref[...], preferred_element_type=F32,
                         precision=lax.Precision.DEFAULT) + bin_ref[...]


def _tc_xs0_body(degp_ref, h_ref, w0_ref, xs_ref, dis_ref):
    n = h_ref.shape[0]
    n_pad = degp_ref.shape[0] // 2
    deg = degp_ref[:n, 0:1] + degp_ref[n_pad:n_pad + n, 0:1] + 1.0
    dis = lax.rsqrt(deg)
    xs_ref[...] = jnp.dot(h_ref[...], w0_ref[...], preferred_element_type=F32,
                          precision=lax.Precision.DEFAULT) * dis
    dis_ref[...] = dis


def _combined_s(sp_ref, n):
    n_pad = sp_ref.shape[0] // 2
    return sp_ref[:n, :] + sp_ref[n_pad:n_pad + n, :]


def _bn_relu_res(sp_ref, xs_ref, h_ref, dis, bcv_ref, gam_ref, bet_ref):
    n = h_ref.shape[0]
    agg = dis * (_combined_s(sp_ref, n) + xs_ref[...]) + bcv_ref[...]
    mean = jnp.mean(agg, axis=0, keepdims=True)
    ctr = agg - mean
    var = jnp.mean(ctr * ctr, axis=0, keepdims=True)
    hn = gam_ref[...] * ctr * lax.rsqrt(var + EPS) + bet_ref[...]
    hn = jnp.maximum(hn, 0.0)
    return h_ref[...] + hn


def _tc_mid_body(sp_ref, xs_ref, h_ref, dis_ref, bcv_ref, gam_ref, bet_ref,
                 wn_ref, hout_ref, xsn_ref):
    dis = dis_ref[...]
    h_new = _bn_relu_res(sp_ref, xs_ref, h_ref, dis, bcv_ref, gam_ref,
                         bet_ref)
    hout_ref[...] = h_new
    xsn_ref[...] = jnp.dot(h_new, wn_ref[...], preferred_element_type=F32,
                           precision=lax.Precision.DEFAULT) * dis


def _tc_final_body(sp_ref, xs_ref, h_ref, dis_ref, bcv_ref, gam_ref, bet_ref,
                   wout_ref, bout_ref, y_ref):
    h_new = _bn_relu_res(sp_ref, xs_ref, h_ref, dis_ref[...], bcv_ref,
                         gam_ref, bet_ref)
    y_ref[...] = jnp.dot(h_new, wout_ref[...], preferred_element_type=F32,
                         precision=lax.Precision.DEFAULT) + bout_ref[...]


def kernel(x, edge_index, W_in, b_in, W_conv, b_conv, gamma, beta,
           W_out, b_out):
    n, _ = x.shape
    d = W_in.shape[1]
    num_layers = W_conv.shape[0]
    e = edge_index.shape[1]

    n_pad = ((n + 64 + 63) // 64) * 64
    cpt = ((-(-e // (CH * NTILES)) + 15) // 16) * 16
    e_pad = cpt * CH * NTILES
    ncht = e_pad // CH

    pad = e_pad - e
    prow = (jnp.arange(pad, dtype=jnp.int32) * 97) % n
    pcol = n + (jnp.arange(pad, dtype=jnp.int32) % (n_pad - n))
    row_p = jnp.concatenate([edge_index[0], prow]).reshape(ncht, CH)
    col_p = jnp.concatenate([edge_index[1], pcol]).reshape(ncht, CH)

    sc_ones = _build_sc_ones_scatter(n_pad, d, ncht, cpt)
    sc_scatter = _build_sc_scatter(n_pad, d, ncht, cpt)

    degp = sc_ones(col_p)

    b_in2 = b_in.reshape(1, d)
    b_out2 = b_out.reshape(1, d)

    h = pl.pallas_call(
        _tc_h0_body,
        out_shape=jax.ShapeDtypeStruct((n, d), F32),
    )(x, W_in, b_in2)
    xs, dis = pl.pallas_call(
        _tc_xs0_body,
        out_shape=[
            jax.ShapeDtypeStruct((n, d), F32),
            jax.ShapeDtypeStruct((n, 1), F32),
        ],
    )(degp, h, W_conv[0])

    for i in range(num_layers):
        sp = sc_scatter(xs, row_p, col_p)
        bcv = b_conv[i].reshape(1, d)
        gam = gamma[i].reshape(1, d)
        bet = beta[i].reshape(1, d)
        cp = pltpu.CompilerParams(vmem_limit_bytes=64 * 1024 * 1024)
        if i < num_layers - 1:
            h, xs = pl.pallas_call(
                _tc_mid_body,
                out_shape=[
                    jax.ShapeDtypeStruct((n, d), F32),
                    jax.ShapeDtypeStruct((n, d), F32),
                ],
                compiler_params=cp,
            )(sp, xs, h, dis, bcv, gam, bet, W_conv[i + 1])
        else:
            out = pl.pallas_call(
                _tc_final_body,
                out_shape=jax.ShapeDtypeStruct((n, d), F32),
                compiler_params=cp,
            )(sp, xs, h, dis, bcv, gam, bet, W_out, b_out2)
    return out
